# baseline (speedup 1.0000x reference)
import jax
import jax.numpy as jnp
from jax.experimental import pallas as pl
from jax.experimental.pallas import tpu as pltpu

_NE = 1024
_ED = 64
_CC = 0.25
_NI = 16
_PX = 1024
_BLK = 1024
_IPB = 16
_GRID = _NI // _IPB


def _vq_body(x_ref, emb_ref, q_ref, idx_ref, loss_ref):
    e = emb_ref[...]
    e2 = e + e
    esq = jnp.sum(e * e, axis=1)
    part = jnp.float32(0.0)
    for s in range(_IPB):
        xT = x_ref[s]
        mm2 = jax.lax.dot_general(e2, xT, (((1,), (0,)), ((), ())),
                                  preferred_element_type=jnp.float32)
        xsq = jnp.sum(xT * xT, axis=0)
        dist = (xsq[None, :] + esq[:, None]) - mm2
        idx = jnp.argmin(dist, axis=0)
        idx_ref[s, 0, :] = idx

        ohT = (jax.lax.broadcasted_iota(jnp.int32, (_NE, _BLK), 0)
               == idx[None, :]).astype(jnp.float32)
        qT = jax.lax.dot_general(e, ohT, (((0,), (0,)), ((), ())),
                                 preferred_element_type=jnp.float32)
        q_ref[s] = qT
        part = part + jnp.sum((qT - xT) ** 2)

    @pl.when(pl.program_id(0) == 0)
    def _init():
        loss_ref[0, 0] = 0.0

    loss_ref[0, 0] += part

    @pl.when(pl.program_id(0) == _GRID - 1)
    def _final():
        loss_ref[0, 0] = loss_ref[0, 0] * (_CC / (_NI * _PX * _ED))


def kernel(inputs, embedding):
    x3 = inputs.reshape(_NI, _ED, _PX)
    q3, idx3, loss = pl.pallas_call(
        _vq_body,
        grid=(_GRID,),
        in_specs=[
            pl.BlockSpec((_IPB, _ED, _BLK), lambda i: (i, 0, 0)),
            pl.BlockSpec((_NE, _ED), lambda i: (0, 0)),
        ],
        out_specs=[
            pl.BlockSpec((_IPB, _ED, _BLK), lambda i: (i, 0, 0)),
            pl.BlockSpec((_IPB, 1, _BLK), lambda i: (i, 0, 0)),
            pl.BlockSpec(block_shape=(1, 1), index_map=lambda i: (0, 0),
                         memory_space=pltpu.SMEM),
        ],
        out_shape=[
            jax.ShapeDtypeStruct((_NI, _ED, _PX), jnp.float32),
            jax.ShapeDtypeStruct((_NI, 1, _BLK), jnp.int32),
            jax.ShapeDtypeStruct((1, 1), jnp.float32),
        ],
    )(x3, embedding)
    return (loss[0, 0],
            q3.reshape(_NI, _ED, 32, 32),
            idx3.reshape(_NI * _PX, 1))


# speedup vs baseline: 1.0268x; 1.0268x over previous
import jax
import jax.numpy as jnp
from jax.experimental import pallas as pl
from jax.experimental.pallas import tpu as pltpu

_NE = 1024
_ED = 64
_CC = 0.25
_NI = 16
_PX = 1024
_BLK = 1024
_IPB = 8
_GRID = _NI // _IPB


def _vq_body(x_ref, emb_ref, q_ref, idx_ref, loss_ref):
    e = emb_ref[...]
    e2 = e + e
    esq = jnp.sum(e * e, axis=1)
    part = jnp.float32(0.0)
    for s in range(_IPB):
        xT = x_ref[s]
        mm2 = jax.lax.dot_general(e2, xT, (((1,), (0,)), ((), ())),
                                  preferred_element_type=jnp.float32)
        xsq = jnp.sum(xT * xT, axis=0)
        dist = (xsq[None, :] + esq[:, None]) - mm2
        idx = jnp.argmin(dist, axis=0)
        idx_ref[s, 0, :] = idx

        ohT = (jax.lax.broadcasted_iota(jnp.int32, (_NE, _BLK), 0)
               == idx[None, :]).astype(jnp.float32)
        qT = jax.lax.dot_general(e, ohT, (((0,), (0,)), ((), ())),
                                 preferred_element_type=jnp.float32)
        q_ref[s] = qT
        part = part + jnp.sum((qT - xT) ** 2)

    @pl.when(pl.program_id(0) == 0)
    def _init():
        loss_ref[0, 0] = 0.0

    loss_ref[0, 0] += part

    @pl.when(pl.program_id(0) == _GRID - 1)
    def _final():
        loss_ref[0, 0] = loss_ref[0, 0] * (_CC / (_NI * _PX * _ED))


def kernel(inputs, embedding):
    x3 = inputs.reshape(_NI, _ED, _PX)
    q3, idx3, loss = pl.pallas_call(
        _vq_body,
        grid=(_GRID,),
        in_specs=[
            pl.BlockSpec((_IPB, _ED, _BLK), lambda i: (i, 0, 0)),
            pl.BlockSpec((_NE, _ED), lambda i: (0, 0)),
        ],
        out_specs=[
            pl.BlockSpec((_IPB, _ED, _BLK), lambda i: (i, 0, 0)),
            pl.BlockSpec((_IPB, 1, _BLK), lambda i: (i, 0, 0)),
            pl.BlockSpec(block_shape=(1, 1), index_map=lambda i: (0, 0),
                         memory_space=pltpu.SMEM),
        ],
        out_shape=[
            jax.ShapeDtypeStruct((_NI, _ED, _PX), jnp.float32),
            jax.ShapeDtypeStruct((_NI, 1, _BLK), jnp.int32),
            jax.ShapeDtypeStruct((1, 1), jnp.float32),
        ],
    )(x3, embedding)
    return (loss[0, 0],
            q3.reshape(_NI, _ED, 32, 32),
            idx3.reshape(_NI * _PX, 1))
